# baseline (speedup 1.0000x reference)
import functools

import jax
import jax.numpy as jnp
from jax import lax
from jax.experimental import pallas as pl
from jax.experimental.pallas import tpu as pltpu
from jax.experimental.pallas import tpu_sc as plsc

_NC = 2
_NS = 16
_NW = _NC * _NS

_B = 4096
_H = 200
_D = 32
_BB = _B // 128


@jax.jit
def _sc_embed(idx_view, weight):
    mesh = plsc.VectorSubcoreMesh(
        core_axis_name="c", subcore_axis_name="s", num_cores=_NC,
        num_subcores=_NS)

    @functools.partial(
        pl.kernel,
        out_type=jax.ShapeDtypeStruct((_H, 4, _BB, 1024), jnp.float32),
        mesh=mesh,
        scratch_types=[
            pltpu.VMEM((_H, 128), jnp.int32),
            pltpu.VMEM((128, _D), jnp.float32),
            pltpu.VMEM((128, _D), jnp.float32),
            pltpu.VMEM((4096,), jnp.float32),
            pltpu.VMEM((4096,), jnp.float32),
            pltpu.SemaphoreType.DMA,
            pltpu.SemaphoreType.DMA,
            pltpu.SemaphoreType.DMA,
            pltpu.SemaphoreType.DMA,
        ],
        compiler_params=pltpu.CompilerParams(
            use_tc_tiling_on_sc=False, needs_layout_passes=False),
    )
    def k(idx_hbm, table_hbm, out_hbm, idx_v, rows0, rows1, tb0, tb1,
          gsem0, gsem1, osem0, osem1):
        w = lax.axis_index("s") * _NC + lax.axis_index("c")
        rows_b = (rows0, rows1)
        tb_b = (tb0, tb1)
        gsem = (gsem0, gsem1)
        osem = (osem0, osem1)
        lanes128 = lax.iota(jnp.int32, 16) * 128

        def gather_start(h, b):
            return pltpu.async_copy(
                table_hbm.at[idx_v.at[h]], rows_b[b], gsem[b])

        def gather_wait(b):
            pltpu.make_async_copy(
                table_hbm.at[pl.ds(0, 128)], rows_b[b], gsem[b]).wait()

        def out_start(h, b):
            for s in range(4):
                pltpu.async_copy(tb_b[b].at[pl.ds(1024 * s, 1024)],
                                 out_hbm.at[h, s, w], osem[b])

        def out_wait(b):
            for s in range(4):
                pltpu.make_async_copy(tb_b[b].at[pl.ds(1024 * s, 1024)],
                                      out_hbm.at[0, s, w], osem[b]).wait()

        def transpose(b):
            rows = rows_b[b]
            tb = tb_b[b]
            for j in range(128):
                for k in range(2):
                    v = rows[j, pl.ds(16 * k, 16)]
                    plsc.store_scatter(tb, [lanes128 + (2048 * k + j)], v)

        def stage(a, carry):
            pltpu.sync_copy(idx_hbm.at[a, w], idx_v.at[pl.ds(8 * a, 8)])
            return carry
        lax.fori_loop(0, _H // 8, stage, 0)

        gather_start(0, 0)

        def pair(u, carry):
            h0 = 2 * u
            gather_start(h0 + 1, 1)
            gather_wait(0)

            @pl.when(u > 0)
            def _():
                out_wait(0)
            transpose(0)
            out_start(h0, 0)
            gather_start(jnp.minimum(h0 + 2, _H - 1), 0)
            gather_wait(1)

            @pl.when(u > 0)
            def _():
                out_wait(1)
            transpose(1)
            out_start(h0 + 1, 1)
            return carry

        lax.fori_loop(0, _H // 2, pair, 0)

        out_wait(0)
        out_wait(1)
        gather_wait(0)

    return k(idx_view, weight)


def kernel(indices, weight):
    idx_view = jnp.transpose(
        jnp.reshape(jnp.transpose(indices), (_H // 8, 8, _BB, 128)),
        (0, 2, 1, 3)).astype(jnp.int32)
    out5 = jnp.reshape(_sc_embed(idx_view, weight), (_H, 4, _BB, 8, 128))
    return jnp.reshape(jnp.transpose(out5, (2, 4, 0, 1, 3)), (_B, _H, _D))


# speedup vs baseline: 1.3979x; 1.3979x over previous
import functools

import jax
import jax.numpy as jnp
from jax import lax
from jax.experimental import pallas as pl
from jax.experimental.pallas import tpu as pltpu
from jax.experimental.pallas import tpu_sc as plsc

_NC = 2
_NS = 16
_NW = _NC * _NS

_B = 4096
_H = 200
_D = 32
_BB = _B // 128


@jax.jit
def _sc_embed(idx_view, weight):
    mesh = plsc.VectorSubcoreMesh(
        core_axis_name="c", subcore_axis_name="s", num_cores=_NC,
        num_subcores=_NS)

    @functools.partial(
        pl.kernel,
        out_type=jax.ShapeDtypeStruct((_H, 4, _BB, 8, 128), jnp.float32),
        mesh=mesh,
        scratch_types=[
            pltpu.VMEM((_H, 128), jnp.int32),
            pltpu.VMEM((128, _D), jnp.float32),
            pltpu.VMEM((128, _D), jnp.float32),
            pltpu.VMEM((_D, 129), jnp.float32),
            pltpu.VMEM((_D, 129), jnp.float32),
            pltpu.SemaphoreType.DMA,
            pltpu.SemaphoreType.DMA,
            pltpu.SemaphoreType.DMA,
            pltpu.SemaphoreType.DMA,
        ],
        compiler_params=pltpu.CompilerParams(
            use_tc_tiling_on_sc=False, needs_layout_passes=False),
    )
    def k(idx_hbm, table_hbm, out_hbm, idx_v, rows0, rows1, tb0, tb1,
          gsem0, gsem1, osem0, osem1):
        w = lax.axis_index("s") * _NC + lax.axis_index("c")
        rows_b = (rows0, rows1)
        tb_b = (tb0, tb1)
        gsem = (gsem0, gsem1)
        osem = (osem0, osem1)
        lanes = lax.iota(jnp.int32, 16)

        def gather_start(h, b):
            return pltpu.async_copy(
                table_hbm.at[idx_v.at[h]], rows_b[b], gsem[b])

        def gather_wait(b):
            pltpu.make_async_copy(
                table_hbm.at[pl.ds(0, 128)], rows_b[b], gsem[b]).wait()

        def out_start(h, b):
            for s in range(4):
                pltpu.async_copy(
                    tb_b[b].at[pl.ds(8 * s, 8), pl.ds(0, 128)],
                    out_hbm.at[h, s, w], osem[b])

        def out_wait(b):
            for s in range(4):
                pltpu.make_async_copy(
                    tb_b[b].at[pl.ds(8 * s, 8), pl.ds(0, 128)],
                    out_hbm.at[0, s, w], osem[b]).wait()

        def transpose(b):
            rows = rows_b[b]
            tb = tb_b[b]
            for j in range(128):
                jv = jnp.full((16,), j, jnp.int32)
                for k in range(2):
                    v = rows[j, pl.ds(16 * k, 16)]
                    plsc.store_scatter(tb, [lanes + 16 * k, jv], v)

        def stage(a, carry):
            pltpu.sync_copy(idx_hbm.at[a, w], idx_v.at[pl.ds(8 * a, 8)])
            return carry
        lax.fori_loop(0, _H // 8, stage, 0)

        gather_start(0, 0)

        def pair(u, carry):
            h0 = 2 * u
            gather_start(h0 + 1, 1)
            gather_wait(0)

            @pl.when(u > 0)
            def _():
                out_wait(0)
            transpose(0)
            out_start(h0, 0)
            gather_start(jnp.minimum(h0 + 2, _H - 1), 0)
            gather_wait(1)

            @pl.when(u > 0)
            def _():
                out_wait(1)
            transpose(1)
            out_start(h0 + 1, 1)
            return carry

        lax.fori_loop(0, _H // 2, pair, 0)

        out_wait(0)
        out_wait(1)
        gather_wait(0)

    return k(idx_view, weight)


def kernel(indices, weight):
    idx_view = jnp.transpose(
        jnp.reshape(jnp.transpose(indices), (_H // 8, 8, _BB, 128)),
        (0, 2, 1, 3)).astype(jnp.int32)
    out5 = _sc_embed(idx_view, weight)
    return jnp.reshape(jnp.transpose(out5, (2, 4, 0, 1, 3)), (_B, _H, _D))


# speedup vs baseline: 1.4296x; 1.0226x over previous
import functools

import jax
import jax.numpy as jnp
from jax import lax
from jax.experimental import pallas as pl
from jax.experimental.pallas import tpu as pltpu
from jax.experimental.pallas import tpu_sc as plsc

_NC = 2
_NS = 16
_NW = _NC * _NS

_B = 4096
_H = 200
_D = 32
_BB = _B // 128


@jax.jit
def _sc_embed(idx_view, weight):
    mesh = plsc.VectorSubcoreMesh(
        core_axis_name="c", subcore_axis_name="s", num_cores=_NC,
        num_subcores=_NS)

    @functools.partial(
        pl.kernel,
        out_type=jax.ShapeDtypeStruct((_H, 4, _BB, 8, 128), jnp.float32),
        mesh=mesh,
        scratch_types=[
            pltpu.VMEM((_H, 128), jnp.int32),
            pltpu.VMEM((128, _D), jnp.float32),
            pltpu.VMEM((128, _D), jnp.float32),
            pltpu.VMEM((_D, 129), jnp.float32),
            pltpu.VMEM((_D, 129), jnp.float32),
            pltpu.SemaphoreType.DMA,
            pltpu.SemaphoreType.DMA,
            pltpu.SemaphoreType.DMA,
            pltpu.SemaphoreType.DMA,
        ],
        compiler_params=pltpu.CompilerParams(
            use_tc_tiling_on_sc=False, needs_layout_passes=False),
    )
    def k(idx_hbm, table_hbm, out_hbm, idx_v, rows0, rows1, tb0, tb1,
          gsem0, gsem1, osem0, osem1):
        w = lax.axis_index("s") * _NC + lax.axis_index("c")
        rows_b = (rows0, rows1)
        tb_b = (tb0, tb1)
        gsem = (gsem0, gsem1)
        osem = (osem0, osem1)
        lanes = lax.iota(jnp.int32, 16)

        def gather_start(h, b):
            return pltpu.async_copy(
                table_hbm.at[idx_v.at[h]], rows_b[b], gsem[b])

        def gather_wait(b):
            pltpu.make_async_copy(
                table_hbm.at[pl.ds(0, 128)], rows_b[b], gsem[b]).wait()

        def out_start(h, b):
            for s in range(4):
                pltpu.async_copy(
                    tb_b[b].at[pl.ds(8 * s, 8), pl.ds(0, 128)],
                    out_hbm.at[h, s, w], osem[b])

        def out_wait(b):
            for s in range(4):
                pltpu.make_async_copy(
                    tb_b[b].at[pl.ds(8 * s, 8), pl.ds(0, 128)],
                    out_hbm.at[0, s, w], osem[b]).wait()

        def transpose(b):
            rows = rows_b[b]
            tb = tb_b[b]

            def body(j, carry):
                jv = jnp.full((16,), j, jnp.int32)
                for k in range(2):
                    v = rows[j, pl.ds(16 * k, 16)]
                    plsc.store_scatter(tb, [lanes + 16 * k, jv], v)
                return carry

            lax.fori_loop(0, 128, body, 0, unroll=4)

        def stage(a, carry):
            pltpu.sync_copy(idx_hbm.at[a, w], idx_v.at[pl.ds(8 * a, 8)])
            return carry
        lax.fori_loop(0, _H // 8, stage, 0)

        gather_start(0, 0)

        def pair(u, carry):
            h0 = 2 * u
            gather_start(h0 + 1, 1)
            gather_wait(0)

            @pl.when(u > 0)
            def _():
                out_wait(0)
            transpose(0)
            out_start(h0, 0)
            gather_start(jnp.minimum(h0 + 2, _H - 1), 0)
            gather_wait(1)

            @pl.when(u > 0)
            def _():
                out_wait(1)
            transpose(1)
            out_start(h0 + 1, 1)
            return carry

        lax.fori_loop(0, _H // 2, pair, 0)

        out_wait(0)
        out_wait(1)
        gather_wait(0)

    return k(idx_view, weight)


def kernel(indices, weight):
    idx_view = jnp.transpose(
        jnp.reshape(jnp.transpose(indices), (_H // 8, 8, _BB, 128)),
        (0, 2, 1, 3)).astype(jnp.int32)
    out5 = _sc_embed(idx_view, weight)
    return jnp.reshape(jnp.transpose(out5, (2, 4, 0, 1, 3)), (_B, _H, _D))


# speedup vs baseline: 1.4637x; 1.0239x over previous
import functools

import jax
import jax.numpy as jnp
from jax import lax
from jax.experimental import pallas as pl
from jax.experimental.pallas import tpu as pltpu
from jax.experimental.pallas import tpu_sc as plsc

_NC = 2
_NS = 16
_NW = _NC * _NS

_B = 4096
_H = 200
_D = 32
_BB = _B // 128


@jax.jit
def _sc_embed(idx_view, weight):
    mesh = plsc.VectorSubcoreMesh(
        core_axis_name="c", subcore_axis_name="s", num_cores=_NC,
        num_subcores=_NS)

    @functools.partial(
        pl.kernel,
        out_type=jax.ShapeDtypeStruct((_H, 4, _BB, 8, 128), jnp.float32),
        mesh=mesh,
        scratch_types=[
            pltpu.VMEM((_H // 8, 8, 128), jnp.int32),
            pltpu.VMEM((128, _D), jnp.float32),
            pltpu.VMEM((128, _D), jnp.float32),
            pltpu.VMEM((_D, 129), jnp.float32),
            pltpu.VMEM((_D, 129), jnp.float32),
            pltpu.SemaphoreType.DMA,
            pltpu.SemaphoreType.DMA,
            pltpu.SemaphoreType.DMA,
            pltpu.SemaphoreType.DMA,
        ],
        compiler_params=pltpu.CompilerParams(
            use_tc_tiling_on_sc=False, needs_layout_passes=False),
    )
    def k(idx_hbm, table_hbm, out_hbm, idx_v, rows0, rows1, tb0, tb1,
          gsem0, gsem1, osem0, osem1):
        w = lax.axis_index("s") * _NC + lax.axis_index("c")
        rows_b = (rows0, rows1)
        tb_b = (tb0, tb1)
        gsem = (gsem0, gsem1)
        osem = (osem0, osem1)
        lanes = lax.iota(jnp.int32, 16)

        def gather_start(h, b):
            return pltpu.async_copy(
                table_hbm.at[idx_v.at[h // 8, h % 8]], rows_b[b], gsem[b])

        def gather_wait(b):
            pltpu.make_async_copy(
                table_hbm.at[pl.ds(0, 128)], rows_b[b], gsem[b]).wait()

        def out_start(h, b):
            for s in range(4):
                pltpu.async_copy(
                    tb_b[b].at[pl.ds(8 * s, 8), pl.ds(0, 128)],
                    out_hbm.at[h, s, w], osem[b])

        def out_wait(b):
            for s in range(4):
                pltpu.make_async_copy(
                    tb_b[b].at[pl.ds(8 * s, 8), pl.ds(0, 128)],
                    out_hbm.at[0, s, w], osem[b]).wait()

        def transpose(b):
            rows = rows_b[b]
            tb = tb_b[b]

            def body(j, carry):
                jv = jnp.full((16,), j, jnp.int32)
                for k in range(2):
                    v = rows[j, pl.ds(16 * k, 16)]
                    plsc.store_scatter(tb, [lanes + 16 * k, jv], v)
                return carry

            lax.fori_loop(0, 128, body, 0, unroll=8)

        pltpu.sync_copy(idx_hbm.at[:, w], idx_v)

        gather_start(0, 0)

        def pair(u, carry):
            h0 = 2 * u
            gather_start(h0 + 1, 1)
            gather_wait(0)

            @pl.when(u > 0)
            def _():
                out_wait(0)
            transpose(0)
            out_start(h0, 0)
            gather_start(jnp.minimum(h0 + 2, _H - 1), 0)
            gather_wait(1)

            @pl.when(u > 0)
            def _():
                out_wait(1)
            transpose(1)
            out_start(h0 + 1, 1)
            return carry

        lax.fori_loop(0, _H // 2, pair, 0)

        out_wait(0)
        out_wait(1)
        gather_wait(0)

    return k(idx_view, weight)


def kernel(indices, weight):
    idx_view = jnp.transpose(
        jnp.reshape(jnp.transpose(indices), (_H // 8, 8, _BB, 128)),
        (0, 2, 1, 3)).astype(jnp.int32)
    out5 = _sc_embed(idx_view, weight)
    return jnp.reshape(jnp.transpose(out5, (2, 4, 0, 1, 3)), (_B, _H, _D))


# speedup vs baseline: 1.4647x; 1.0007x over previous
import functools

import jax
import jax.numpy as jnp
from jax import lax
from jax.experimental import pallas as pl
from jax.experimental.pallas import tpu as pltpu
from jax.experimental.pallas import tpu_sc as plsc

_NC = 2
_NS = 16
_NW = _NC * _NS

_B = 4096
_H = 200
_D = 32
_BB = _B // 128


@jax.jit
def _sc_embed(idx_view, weight):
    mesh = plsc.VectorSubcoreMesh(
        core_axis_name="c", subcore_axis_name="s", num_cores=_NC,
        num_subcores=_NS)

    @functools.partial(
        pl.kernel,
        out_type=jax.ShapeDtypeStruct((_H, 4, _BB, 8, 128), jnp.float32),
        mesh=mesh,
        scratch_types=[
            pltpu.VMEM((_H // 8, 8, 128), jnp.int32),
            pltpu.VMEM((128, _D), jnp.float32),
            pltpu.VMEM((128, _D), jnp.float32),
            pltpu.VMEM((_D, 129), jnp.float32),
            pltpu.VMEM((_D, 129), jnp.float32),
            pltpu.SemaphoreType.DMA,
            pltpu.SemaphoreType.DMA,
            pltpu.SemaphoreType.DMA,
            pltpu.SemaphoreType.DMA,
        ],
        compiler_params=pltpu.CompilerParams(
            use_tc_tiling_on_sc=False, needs_layout_passes=False),
    )
    def k(idx_hbm, table_hbm, out_hbm, idx_v, rows0, rows1, tb0, tb1,
          gsem0, gsem1, osem0, osem1):
        w = lax.axis_index("s") * _NC + lax.axis_index("c")
        rows_b = (rows0, rows1)
        tb_b = (tb0, tb1)
        gsem = (gsem0, gsem1)
        osem = (osem0, osem1)
        lanes = lax.iota(jnp.int32, 16)

        def gather_start(h, b):
            return pltpu.async_copy(
                table_hbm.at[idx_v.at[h // 8, h % 8]], rows_b[b], gsem[b])

        def gather_wait(b):
            pltpu.make_async_copy(
                table_hbm.at[pl.ds(0, 128)], rows_b[b], gsem[b]).wait()

        def out_start(h, b):
            for s in range(4):
                pltpu.async_copy(
                    tb_b[b].at[pl.ds(8 * s, 8), pl.ds(0, 128)],
                    out_hbm.at[h, s, w], osem[b])

        def out_wait(b):
            for s in range(4):
                pltpu.make_async_copy(
                    tb_b[b].at[pl.ds(8 * s, 8), pl.ds(0, 128)],
                    out_hbm.at[0, s, w], osem[b]).wait()

        def transpose(b):
            rows = rows_b[b]
            tb = tb_b[b]

            def body(j, carry):
                jv = jnp.full((16,), j, jnp.int32)
                for k in range(2):
                    v = rows[j, pl.ds(16 * k, 16)]
                    plsc.store_scatter(tb, [lanes + 16 * k, jv], v)
                return carry

            lax.fori_loop(0, 128, body, 0, unroll=16)

        pltpu.sync_copy(idx_hbm.at[:, w], idx_v)

        gather_start(0, 0)

        def pair(u, carry):
            h0 = 2 * u
            gather_start(h0 + 1, 1)
            gather_wait(0)

            @pl.when(u > 0)
            def _():
                out_wait(0)
            transpose(0)
            out_start(h0, 0)
            gather_start(jnp.minimum(h0 + 2, _H - 1), 0)
            gather_wait(1)

            @pl.when(u > 0)
            def _():
                out_wait(1)
            transpose(1)
            out_start(h0 + 1, 1)
            return carry

        lax.fori_loop(0, _H // 2, pair, 0)

        out_wait(0)
        out_wait(1)
        gather_wait(0)

    return k(idx_view, weight)


def kernel(indices, weight):
    idx_view = jnp.transpose(
        jnp.reshape(jnp.transpose(indices), (_H // 8, 8, _BB, 128)),
        (0, 2, 1, 3)).astype(jnp.int32)
    out5 = _sc_embed(idx_view, weight)
    return jnp.reshape(jnp.transpose(out5, (2, 4, 0, 1, 3)), (_B, _H, _D))
